# baseline (speedup 1.0000x reference)
import jax
import jax.numpy as jnp
from jax.experimental import pallas as pl
from jax.experimental.pallas import tpu as pltpu

_CAP_ROWS = 262144
_ROW_DIM = 128
_BLOCK = 16384


def _pushback_body(end_ref, data_ref, buf_ref, out_ref):
    out_ref[...] = buf_ref[...]
    i = pl.program_id(0)
    local = end_ref[0] - i * _BLOCK

    @pl.when((local >= 0) & (local < _BLOCK))
    def _():
        out_ref[pl.ds(local, 1), :] = data_ref[...]


def kernel(data, buffer, start_included, end_excluded, length):
    end = jnp.asarray(end_excluded, jnp.int32).reshape(1)
    data2 = data.reshape(1, _ROW_DIM)
    return pl.pallas_call(
        _pushback_body,
        grid=(_CAP_ROWS // _BLOCK,),
        in_specs=[
            pl.BlockSpec(memory_space=pltpu.SMEM),
            pl.BlockSpec((1, _ROW_DIM), lambda i: (0, 0)),
            pl.BlockSpec((_BLOCK, _ROW_DIM), lambda i: (i, 0)),
        ],
        out_specs=pl.BlockSpec((_BLOCK, _ROW_DIM), lambda i: (i, 0)),
        out_shape=jax.ShapeDtypeStruct((_CAP_ROWS, _ROW_DIM), jnp.float32),
        compiler_params=pltpu.CompilerParams(
            dimension_semantics=("parallel",),
            vmem_limit_bytes=64 * 1024 * 1024,
        ),
    )(end, data2, buffer)
